# baseline (speedup 1.0000x reference)
import jax
import jax.numpy as jnp
from jax.experimental import pallas as pl
from jax.experimental.pallas import tpu as pltpu

B, S, K, D = 1024, 20, 100, 32
BT = 128
NT = B // BT
C = K * BT
CH = 256
NC = C // CH


def _entity_kernel(x_ref, m_ref, keys_ref, Ut_ref, Vt_ref, Wt_ref, out_ref,
                   kv_ref, hu_ref, ha_ref, hb_ref):
    Ut = Ut_ref[...]
    Wt = Wt_ref[...]
    ones_row = jnp.ones((1, D), dtype=jnp.float32)

    kv_ref[...] = jnp.dot(Vt_ref[...], keys_ref[0],
                          preferred_element_type=jnp.float32)
    ha_ref[...] = jnp.zeros((D, C), dtype=jnp.float32)

    def substep(t, src, dst, final):
        x = x_ref[t]
        m = m_ref[t]
        xW = jnp.dot(Wt, x, preferred_element_type=jnp.float32)
        rep = CH // BT
        x2 = jnp.concatenate([x] * rep, axis=1)
        m2 = jnp.concatenate([m] * rep, axis=1)
        notm2 = 1.0 - m2
        xw2 = jnp.concatenate([xW] * rep, axis=1)
        hu_ref[...] = jnp.dot(Ut, src[...],
                              preferred_element_type=jnp.float32) + kv_ref[...]
        for c in range(NC):
            sl = slice(c * CH, (c + 1) * CH)
            Hc = src[:, sl]
            Kc = keys_ref[0, :, sl]
            g = jax.nn.sigmoid(
                jnp.dot(ones_row, x2 * (Hc + Kc),
                        preferred_element_type=jnp.float32))
            ht = jnp.maximum(hu_ref[:, sl] + xw2, 0.0)
            u = Hc + g * ht
            sq = jnp.dot(ones_row, u * u,
                         preferred_element_type=jnp.float32)
            scale = jax.lax.rsqrt(jnp.maximum(sq, 1e-12))
            res = Hc * notm2 + (m2 * scale) * u
            if final:
                for j in range(rep):
                    k = c * rep + j
                    out_ref[:, k, :] = jnp.swapaxes(
                        res[:, j * BT:(j + 1) * BT], 0, 1)
            else:
                dst[:, sl] = res

    def double_step(i, carry):
        substep(2 * i, ha_ref, hb_ref, False)
        substep(2 * i + 1, hb_ref, ha_ref, False)
        return carry

    jax.lax.fori_loop(0, S // 2 - 1, double_step, 0)
    substep(S - 2, ha_ref, hb_ref, False)
    substep(S - 1, hb_ref, None, True)


@jax.jit
def kernel(encoded_sents, mask, keys, U, V, W):
    x_all = jnp.transpose(encoded_sents, (1, 2, 0))
    m_all = jnp.swapaxes(mask, 0, 1).astype(jnp.float32)[:, None, :]
    keysR = jnp.transpose(keys, (2, 1, 0))
    keysR = keysR.reshape(D, K, NT, BT).transpose(2, 0, 1, 3).reshape(NT, D, C)

    out = pl.pallas_call(
        _entity_kernel,
        grid=(NT,),
        in_specs=[
            pl.BlockSpec((S, D, BT), lambda i: (0, 0, i)),
            pl.BlockSpec((S, 1, BT), lambda i: (0, 0, i)),
            pl.BlockSpec((1, D, C), lambda i: (i, 0, 0)),
            pl.BlockSpec((D, D), lambda i: (0, 0)),
            pl.BlockSpec((D, D), lambda i: (0, 0)),
            pl.BlockSpec((D, D), lambda i: (0, 0)),
        ],
        out_specs=pl.BlockSpec((BT, K, D), lambda i: (i, 0, 0)),
        out_shape=jax.ShapeDtypeStruct((B, K, D), jnp.float32),
        scratch_shapes=[
            pltpu.VMEM((D, C), jnp.float32),
            pltpu.VMEM((D, C), jnp.float32),
            pltpu.VMEM((D, C), jnp.float32),
            pltpu.VMEM((D, C), jnp.float32),
        ],
    )(x_all, m_all, keysR, U.T, V.T, W.T)
    return out


# speedup vs baseline: 1.0685x; 1.0685x over previous
import jax
import jax.numpy as jnp
from jax.experimental import pallas as pl
from jax.experimental.pallas import tpu as pltpu

B, S, K, D = 1024, 20, 100, 32
BT = 128
NT = B // BT
C = K * BT
CH = 256
NC = C // CH


def _entity_kernel(x_ref, m_ref, keys_ref, Ut_ref, Vt_ref, Wt_ref, out_ref,
                   kv_ref, hu_ref, hb_ref):
    Ut = Ut_ref[...]
    Wt = Wt_ref[...]
    ones_row = jnp.ones((1, D), dtype=jnp.float32)

    kv_ref[...] = jnp.dot(Vt_ref[...], keys_ref[0],
                          preferred_element_type=jnp.float32)
    out_ref[0] = jnp.zeros((D, C), dtype=jnp.float32)

    def substep(t, src, dst):
        x = jnp.swapaxes(x_ref[:, t, :], 0, 1)
        m = m_ref[t]
        xW = jnp.dot(Wt, x, preferred_element_type=jnp.float32)
        rep = CH // BT
        x2 = jnp.concatenate([x] * rep, axis=1)
        m2 = jnp.concatenate([m] * rep, axis=1)
        notm2 = 1.0 - m2
        xw2 = jnp.concatenate([xW] * rep, axis=1)
        Hfull = src[0] if src is out_ref else src[...]
        hu_ref[...] = jnp.dot(Ut, Hfull,
                              preferred_element_type=jnp.float32) + kv_ref[...]
        for c in range(NC):
            sl = slice(c * CH, (c + 1) * CH)
            if src is out_ref:
                Hc = src[0, :, sl]
            else:
                Hc = src[:, sl]
            Kc = keys_ref[0, :, sl]
            g = jax.nn.sigmoid(
                jnp.dot(ones_row, x2 * (Hc + Kc),
                        preferred_element_type=jnp.float32))
            ht = jnp.maximum(hu_ref[:, sl] + xw2, 0.0)
            u = Hc + g * ht
            sq = jnp.dot(ones_row, u * u,
                         preferred_element_type=jnp.float32)
            scale = jax.lax.rsqrt(jnp.maximum(sq, 1e-12))
            res = Hc * notm2 + (m2 * scale) * u
            if dst is out_ref:
                dst[0, :, sl] = res
            else:
                dst[:, sl] = res

    def double_step(i, carry):
        substep(2 * i, out_ref, hb_ref)
        substep(2 * i + 1, hb_ref, out_ref)
        return carry

    jax.lax.fori_loop(0, S // 2, double_step, 0)


@jax.jit
def kernel(encoded_sents, mask, keys, U, V, W):
    x_all = encoded_sents
    m_all = jnp.swapaxes(mask, 0, 1).astype(jnp.float32)[:, None, :]
    keysR = jnp.transpose(keys, (2, 1, 0))
    keysR = keysR.reshape(D, K, NT, BT).transpose(2, 0, 1, 3).reshape(NT, D, C)

    hT = pl.pallas_call(
        _entity_kernel,
        grid=(NT,),
        in_specs=[
            pl.BlockSpec((BT, S, D), lambda i: (i, 0, 0)),
            pl.BlockSpec((S, 1, BT), lambda i: (0, 0, i)),
            pl.BlockSpec((1, D, C), lambda i: (i, 0, 0)),
            pl.BlockSpec((D, D), lambda i: (0, 0)),
            pl.BlockSpec((D, D), lambda i: (0, 0)),
            pl.BlockSpec((D, D), lambda i: (0, 0)),
        ],
        out_specs=pl.BlockSpec((1, D, C), lambda i: (i, 0, 0)),
        out_shape=jax.ShapeDtypeStruct((NT, D, C), jnp.float32),
        scratch_shapes=[
            pltpu.VMEM((D, C), jnp.float32),
            pltpu.VMEM((D, C), jnp.float32),
            pltpu.VMEM((D, C), jnp.float32),
        ],
    )(x_all, m_all, keysR, U.T, V.T, W.T)

    out = hT.reshape(NT, D, K, BT).transpose(0, 3, 2, 1).reshape(B, K, D)
    return out


# speedup vs baseline: 1.1022x; 1.0316x over previous
import jax
import jax.numpy as jnp
from jax.experimental import pallas as pl
from jax.experimental.pallas import tpu as pltpu

B, S, K, D = 1024, 20, 100, 32
BT = 128
NT = B // BT
C = K * BT
CH = 256
NC = C // CH


def _entity_kernel(x_ref, m_ref, keys_ref, Ut_ref, Vt_ref, Wt_ref, out_ref,
                   kv_ref, hu_ref, hb_ref):
    Ut = Ut_ref[...]
    Wt = Wt_ref[...]
    ones_row = jnp.ones((1, D), dtype=jnp.float32)

    kv_ref[...] = jnp.dot(Vt_ref[...], keys_ref[0],
                          preferred_element_type=jnp.float32)
    out_ref[0] = jnp.zeros((D, C), dtype=jnp.float32)

    def substep(t, src, dst):
        x = x_ref[t]
        m = m_ref[t]
        xW = jnp.dot(Wt, x, preferred_element_type=jnp.float32)
        rep = CH // BT
        x2 = jnp.concatenate([x] * rep, axis=1)
        m2 = jnp.concatenate([m] * rep, axis=1)
        notm2 = 1.0 - m2
        xw2 = jnp.concatenate([xW] * rep, axis=1)
        Hfull = src[0] if src is out_ref else src[...]
        hu_ref[...] = jnp.dot(Ut, Hfull,
                              preferred_element_type=jnp.float32) + kv_ref[...]
        for c in range(NC):
            sl = slice(c * CH, (c + 1) * CH)
            if src is out_ref:
                Hc = src[0, :, sl]
            else:
                Hc = src[:, sl]
            Kc = keys_ref[0, :, sl]
            g = jax.nn.sigmoid(
                jnp.dot(ones_row, x2 * (Hc + Kc),
                        preferred_element_type=jnp.float32))
            ht = jnp.maximum(hu_ref[:, sl] + xw2, 0.0)
            u = Hc + g * ht
            sq = jnp.dot(ones_row, u * u,
                         preferred_element_type=jnp.float32)
            scale = jax.lax.rsqrt(jnp.maximum(sq, 1e-12))
            res = Hc * notm2 + (m2 * scale) * u
            if dst is out_ref:
                dst[0, :, sl] = res
            else:
                dst[:, sl] = res

    def double_step(i, carry):
        substep(2 * i, out_ref, hb_ref)
        substep(2 * i + 1, hb_ref, out_ref)
        return carry

    jax.lax.fori_loop(0, S // 2, double_step, 0)


@jax.jit
def kernel(encoded_sents, mask, keys, U, V, W):
    x_all = jnp.transpose(encoded_sents, (1, 2, 0))
    m_all = jnp.swapaxes(mask, 0, 1).astype(jnp.float32)[:, None, :]
    keysR = keys.reshape(NT, BT, K, D).transpose(0, 3, 2, 1).reshape(NT, D, C)

    hT = pl.pallas_call(
        _entity_kernel,
        grid=(NT,),
        in_specs=[
            pl.BlockSpec((S, D, BT), lambda i: (0, 0, i)),
            pl.BlockSpec((S, 1, BT), lambda i: (0, 0, i)),
            pl.BlockSpec((1, D, C), lambda i: (i, 0, 0)),
            pl.BlockSpec((D, D), lambda i: (0, 0)),
            pl.BlockSpec((D, D), lambda i: (0, 0)),
            pl.BlockSpec((D, D), lambda i: (0, 0)),
        ],
        out_specs=pl.BlockSpec((1, D, C), lambda i: (i, 0, 0)),
        out_shape=jax.ShapeDtypeStruct((NT, D, C), jnp.float32),
        scratch_shapes=[
            pltpu.VMEM((D, C), jnp.float32),
            pltpu.VMEM((D, C), jnp.float32),
            pltpu.VMEM((D, C), jnp.float32),
        ],
    )(x_all, m_all, keysR, U.T, V.T, W.T)

    out = hT.reshape(NT, D, K, BT).transpose(0, 3, 2, 1).reshape(B, K, D)
    return out


# speedup vs baseline: 1.1140x; 1.0106x over previous
import jax
import jax.numpy as jnp
from jax.experimental import pallas as pl
from jax.experimental.pallas import tpu as pltpu

B, S, K, D = 1024, 20, 100, 32
BT = 128
NT = B // BT
C = K * BT
CH = 256
NC = C // CH


def _entity_kernel(x_ref, m_ref, keys_ref, Ut_ref, Vt_ref, Wt_ref, out_ref,
                   kv_ref, hu_ref, hb_ref):
    Ut = Ut_ref[...]
    Wt = Wt_ref[...]
    ones_row = jnp.ones((1, D), dtype=jnp.float32)

    kv_ref[...] = jnp.dot(Vt_ref[...], keys_ref[0],
                          preferred_element_type=jnp.float32)
    out_ref[0] = jnp.zeros((D, C), dtype=jnp.float32)

    def substep(t, src, dst):
        x = x_ref[t]
        m = m_ref[t]
        xW = jnp.dot(Wt, x, preferred_element_type=jnp.float32)
        rep = CH // BT
        x2 = jnp.concatenate([x] * rep, axis=1)
        m2 = jnp.concatenate([m] * rep, axis=1)
        notm2 = 1.0 - m2
        xw2 = jnp.concatenate([xW] * rep, axis=1)
        Hfull = src[0] if src is out_ref else src[...]
        hu_ref[...] = jnp.dot(Ut, Hfull,
                              preferred_element_type=jnp.float32) + kv_ref[...]
        for c in range(NC):
            sl = slice(c * CH, (c + 1) * CH)
            if src is out_ref:
                Hc = src[0, :, sl]
            else:
                Hc = src[:, sl]
            Kc = keys_ref[0, :, sl]
            g = jax.nn.sigmoid(
                jnp.sum(x2 * (Hc + Kc), axis=0, keepdims=True))
            ht = jnp.maximum(hu_ref[:, sl] + xw2, 0.0)
            u = Hc + g * ht
            sq = jnp.dot(ones_row, u * u,
                         preferred_element_type=jnp.float32)
            scale = jax.lax.rsqrt(jnp.maximum(sq, 1e-12))
            res = Hc * notm2 + (m2 * scale) * u
            if dst is out_ref:
                dst[0, :, sl] = res
            else:
                dst[:, sl] = res

    def double_step(i, carry):
        substep(2 * i, out_ref, hb_ref)
        substep(2 * i + 1, hb_ref, out_ref)
        return carry

    jax.lax.fori_loop(0, S // 2, double_step, 0)


@jax.jit
def kernel(encoded_sents, mask, keys, U, V, W):
    x_all = jnp.transpose(encoded_sents, (1, 2, 0))
    m_all = jnp.swapaxes(mask, 0, 1).astype(jnp.float32)[:, None, :]
    keysR = keys.reshape(NT, BT, K, D).transpose(0, 3, 2, 1).reshape(NT, D, C)

    hT = pl.pallas_call(
        _entity_kernel,
        grid=(NT,),
        in_specs=[
            pl.BlockSpec((S, D, BT), lambda i: (0, 0, i)),
            pl.BlockSpec((S, 1, BT), lambda i: (0, 0, i)),
            pl.BlockSpec((1, D, C), lambda i: (i, 0, 0)),
            pl.BlockSpec((D, D), lambda i: (0, 0)),
            pl.BlockSpec((D, D), lambda i: (0, 0)),
            pl.BlockSpec((D, D), lambda i: (0, 0)),
        ],
        out_specs=pl.BlockSpec((1, D, C), lambda i: (i, 0, 0)),
        out_shape=jax.ShapeDtypeStruct((NT, D, C), jnp.float32),
        scratch_shapes=[
            pltpu.VMEM((D, C), jnp.float32),
            pltpu.VMEM((D, C), jnp.float32),
            pltpu.VMEM((D, C), jnp.float32),
        ],
    )(x_all, m_all, keysR, U.T, V.T, W.T)

    out = hT.reshape(NT, D, K, BT).transpose(0, 3, 2, 1).reshape(B, K, D)
    return out
